# baseline (speedup 1.0000x reference)
import functools

import jax
import jax.numpy as jnp
from jax import lax
from jax.experimental import pallas as pl
from jax.experimental.pallas import tpu as pltpu
from jax.experimental.pallas import tpu_sc as plsc

G = 8
GS = 12
LANES = 16
CH = 1536


def _make_sc_kernel(n_rows, hw):
    info = plsc.get_sparse_core_info()
    nc = info.num_cores
    n_chunks = hw // CH
    assert n_chunks % 2 == 0
    mesh = plsc.VectorSubcoreMesh(core_axis_name="c", subcore_axis_name="s")

    @functools.partial(
        pl.kernel,
        mesh=mesh,
        out_type=jax.ShapeDtypeStruct((n_rows, GS, hw), jnp.float32),
        scratch_types=[
            pltpu.VMEM((GS, CH), jnp.float32),
            pltpu.VMEM((GS, CH), jnp.float32),
            pltpu.VMEM((GS, CH), jnp.float32),
            pltpu.VMEM((GS, CH), jnp.float32),
            pltpu.VMEM((2 * GS, LANES), jnp.float32),
            pltpu.SemaphoreType.DMA,
            pltpu.SemaphoreType.DMA,
            pltpu.SemaphoreType.DMA,
            pltpu.SemaphoreType.DMA,
        ],
    )
    def sc_kernel(x_hbm, w_hbm, out_hbm, in0, in1, ob0, ob1, w_v,
                  si0, si1, so0, so1):
        wid = lax.axis_index("s") * nc + lax.axis_index("c")
        g = lax.rem(wid, G)
        pltpu.sync_copy(w_hbm.at[g], w_v)
        w0 = [w_v[j] for j in range(GS)]
        w1 = [w_v[GS + j] for j in range(GS)]
        in_bufs, out_bufs = (in0, in1), (ob0, ob1)
        sin, sout = (si0, si1), (so0, so1)

        def in_slice(ci):
            return x_hbm.at[wid, :, pl.ds(ci * CH, CH)]

        def out_slice(ci):
            return out_hbm.at[wid, :, pl.ds(ci * CH, CH)]

        pltpu.make_async_copy(in_slice(0), in0, si0).start()
        pltpu.make_async_copy(in_slice(1), in1, si1).start()

        def step(i, carry):
            for b in range(2):
                ci = 2 * i + b
                ibuf, obuf = in_bufs[b], out_bufs[b]

                @pl.when(ci >= 2)
                def _wait_out():
                    pltpu.make_async_copy(obuf, out_slice(ci - 2), sout[b]).wait()

                pltpu.make_async_copy(in_slice(ci), ibuf, sin[b]).wait()

                @plsc.parallel_loop(0, CH // LANES, unroll=4)
                def pix_body(p):
                    po = p * LANES
                    vals = [ibuf[j, pl.ds(po, LANES)] for j in range(GS)]
                    m1 = jnp.maximum(vals[0], vals[1])
                    m2 = jnp.minimum(vals[0], vals[1])
                    for j in range(2, GS):
                        v = vals[j]
                        m2 = jnp.maximum(m2, jnp.minimum(m1, v))
                        m1 = jnp.maximum(m1, v)
                    for j in range(GS):
                        obuf[j, pl.ds(po, LANES)] = vals[j] + w0[j] * m1 + w1[j] * m2

                @pl.when(ci + 2 < n_chunks)
                def _next_in():
                    pltpu.make_async_copy(in_slice(ci + 2), ibuf, sin[b]).start()

                pltpu.make_async_copy(obuf, out_slice(ci), sout[b]).start()
            return carry

        lax.fori_loop(0, n_chunks // 2, step, 0)
        pltpu.make_async_copy(ob0, out_slice(n_chunks - 2), so0).wait()
        pltpu.make_async_copy(ob1, out_slice(n_chunks - 1), so1).wait()

    return sc_kernel


def kernel(input_tensor, weight):
    n, c, h, w = input_tensor.shape
    hw = h * w
    x3 = input_tensor.reshape(n * G, GS, hw)
    wr = weight.reshape(G, GS, 2)
    wcat = jnp.concatenate([wr[:, :, 0], wr[:, :, 1]], axis=1)
    wb = jnp.broadcast_to(wcat[:, :, None], (G, 2 * GS, LANES))
    out3 = _make_sc_kernel(n * G, hw)(x3, wb)
    return out3.reshape(n, c, h, w)


# speedup vs baseline: 1.0652x; 1.0652x over previous
import functools

import jax
import jax.numpy as jnp
from jax import lax
from jax.experimental import pallas as pl
from jax.experimental.pallas import tpu as pltpu
from jax.experimental.pallas import tpu_sc as plsc

G = 8
GS = 12
LANES = 16
CH = 1536


def _make_sc_kernel(n_rows, hw):
    info = plsc.get_sparse_core_info()
    nc = info.num_cores
    n_chunks = hw // CH
    assert n_chunks % 2 == 0
    mesh = plsc.VectorSubcoreMesh(core_axis_name="c", subcore_axis_name="s")

    @functools.partial(
        pl.kernel,
        mesh=mesh,
        out_type=jax.ShapeDtypeStruct((n_rows, GS, hw), jnp.float32),
        scratch_types=[
            pltpu.VMEM((GS, CH), jnp.float32),
            pltpu.VMEM((GS, CH), jnp.float32),
            pltpu.VMEM((GS, CH), jnp.float32),
            pltpu.VMEM((GS, CH), jnp.float32),
            pltpu.VMEM((2 * GS, LANES), jnp.float32),
            pltpu.SemaphoreType.DMA,
            pltpu.SemaphoreType.DMA,
            pltpu.SemaphoreType.DMA,
            pltpu.SemaphoreType.DMA,
        ],
    )
    def sc_kernel(x_hbm, w_hbm, out_hbm, in0, in1, ob0, ob1, w_v,
                  si0, si1, so0, so1):
        wid = lax.axis_index("s") * nc + lax.axis_index("c")
        g = lax.rem(wid, G)
        pltpu.sync_copy(w_hbm.at[g], w_v)
        w0 = [w_v[j] for j in range(GS)]
        w1 = [w_v[GS + j] for j in range(GS)]
        in_bufs, out_bufs = (in0, in1), (ob0, ob1)
        sin, sout = (si0, si1), (so0, so1)

        def in_slice(ci):
            return x_hbm.at[wid, :, pl.ds(ci * CH, CH)]

        def out_slice(ci):
            return out_hbm.at[wid, :, pl.ds(ci * CH, CH)]

        pltpu.make_async_copy(in_slice(0), in0, si0).start()
        pltpu.make_async_copy(in_slice(1), in1, si1).start()

        def step(i, carry):
            for b in range(2):
                ci = 2 * i + b
                ibuf, obuf = in_bufs[b], out_bufs[b]

                @pl.when(ci >= 2)
                def _wait_out():
                    pltpu.make_async_copy(obuf, out_slice(ci - 2), sout[b]).wait()

                pltpu.make_async_copy(in_slice(ci), ibuf, sin[b]).wait()

                @plsc.parallel_loop(0, CH // LANES, unroll=2)
                def pix_body(p):
                    po = p * LANES
                    vals = [ibuf[j, pl.ds(po, LANES)] for j in range(GS)]
                    m1 = jnp.maximum(vals[0], vals[1])
                    m2 = jnp.minimum(vals[0], vals[1])
                    for j in range(2, GS):
                        v = vals[j]
                        m2 = jnp.maximum(m2, jnp.minimum(m1, v))
                        m1 = jnp.maximum(m1, v)
                    for j in range(GS):
                        obuf[j, pl.ds(po, LANES)] = vals[j] + w0[j] * m1 + w1[j] * m2

                @pl.when(ci + 2 < n_chunks)
                def _next_in():
                    pltpu.make_async_copy(in_slice(ci + 2), ibuf, sin[b]).start()

                pltpu.make_async_copy(obuf, out_slice(ci), sout[b]).start()
            return carry

        lax.fori_loop(0, n_chunks // 2, step, 0)
        pltpu.make_async_copy(ob0, out_slice(n_chunks - 2), so0).wait()
        pltpu.make_async_copy(ob1, out_slice(n_chunks - 1), so1).wait()

    return sc_kernel


def kernel(input_tensor, weight):
    n, c, h, w = input_tensor.shape
    hw = h * w
    x3 = input_tensor.reshape(n * G, GS, hw)
    wr = weight.reshape(G, GS, 2)
    wcat = jnp.concatenate([wr[:, :, 0], wr[:, :, 1]], axis=1)
    wb = jnp.broadcast_to(wcat[:, :, None], (G, 2 * GS, LANES))
    out3 = _make_sc_kernel(n * G, hw)(x3, wb)
    return out3.reshape(n, c, h, w)


# speedup vs baseline: 3.2175x; 3.0205x over previous
import functools

import jax
import jax.numpy as jnp
from jax import lax
from jax.experimental import pallas as pl
from jax.experimental.pallas import tpu as pltpu
from jax.experimental.pallas import tpu_sc as plsc

G = 8
GS = 12
LANES = 16
ROWS = 4


def _make_sc_kernel(n_rows, h, w):
    info = plsc.get_sparse_core_info()
    nc = info.num_cores
    n_chunks = h // ROWS
    assert n_chunks % 2 == 0
    mesh = plsc.VectorSubcoreMesh(core_axis_name="c", subcore_axis_name="s")

    @functools.partial(
        pl.kernel,
        mesh=mesh,
        out_type=jax.ShapeDtypeStruct((n_rows, GS, h, w), jnp.float32),
        scratch_types=[
            pltpu.VMEM((GS, ROWS, w), jnp.float32),
            pltpu.VMEM((GS, ROWS, w), jnp.float32),
            pltpu.VMEM((GS, ROWS, w), jnp.float32),
            pltpu.VMEM((GS, ROWS, w), jnp.float32),
            pltpu.VMEM((2 * GS, LANES), jnp.float32),
            pltpu.SemaphoreType.DMA,
            pltpu.SemaphoreType.DMA,
            pltpu.SemaphoreType.DMA,
            pltpu.SemaphoreType.DMA,
        ],
    )
    def sc_kernel(x_hbm, w_hbm, out_hbm, in0, in1, ob0, ob1, w_v,
                  si0, si1, so0, so1):
        wid = lax.axis_index("s") * nc + lax.axis_index("c")
        g = lax.rem(wid, G)
        pltpu.sync_copy(w_hbm.at[g], w_v)
        w0 = [w_v[j] for j in range(GS)]
        w1 = [w_v[GS + j] for j in range(GS)]
        in_bufs, out_bufs = (in0, in1), (ob0, ob1)
        sin, sout = (si0, si1), (so0, so1)

        def in_slice(ci):
            return x_hbm.at[wid, :, pl.ds(ci * ROWS, ROWS), :]

        def out_slice(ci):
            return out_hbm.at[wid, :, pl.ds(ci * ROWS, ROWS), :]

        pltpu.make_async_copy(in_slice(0), in0, si0).start()
        pltpu.make_async_copy(in_slice(1), in1, si1).start()

        def step(i, carry):
            for b in range(2):
                ci = 2 * i + b
                ibuf, obuf = in_bufs[b], out_bufs[b]

                @pl.when(ci >= 2)
                def _wait_out():
                    pltpu.make_async_copy(obuf, out_slice(ci - 2), sout[b]).wait()

                pltpu.make_async_copy(in_slice(ci), ibuf, sin[b]).wait()

                @plsc.parallel_loop(0, w // LANES, unroll=2)
                def pix_body(p):
                    po = p * LANES
                    for r in range(ROWS):
                        vals = [ibuf[j, r, pl.ds(po, LANES)] for j in range(GS)]
                        m1 = jnp.maximum(vals[0], vals[1])
                        m2 = jnp.minimum(vals[0], vals[1])
                        for j in range(2, GS):
                            v = vals[j]
                            m2 = jnp.maximum(m2, jnp.minimum(m1, v))
                            m1 = jnp.maximum(m1, v)
                        for j in range(GS):
                            obuf[j, r, pl.ds(po, LANES)] = (
                                vals[j] + w0[j] * m1 + w1[j] * m2)

                @pl.when(ci + 2 < n_chunks)
                def _next_in():
                    pltpu.make_async_copy(in_slice(ci + 2), ibuf, sin[b]).start()

                pltpu.make_async_copy(obuf, out_slice(ci), sout[b]).start()
            return carry

        lax.fori_loop(0, n_chunks // 2, step, 0)
        pltpu.make_async_copy(ob0, out_slice(n_chunks - 2), so0).wait()
        pltpu.make_async_copy(ob1, out_slice(n_chunks - 1), so1).wait()

    return sc_kernel


def kernel(input_tensor, weight):
    n, c, h, w = input_tensor.shape
    x4 = input_tensor.reshape(n * G, GS, h, w)
    wr = weight.reshape(G, GS, 2)
    wcat = jnp.concatenate([wr[:, :, 0], wr[:, :, 1]], axis=1)
    wb = jnp.broadcast_to(wcat[:, :, None], (G, 2 * GS, LANES))
    out4 = _make_sc_kernel(n * G, h, w)(x4, wb)
    return out4.reshape(n, c, h, w)


# speedup vs baseline: 4.3778x; 1.3606x over previous
import functools

import jax
import jax.numpy as jnp
from jax import lax
from jax.experimental import pallas as pl
from jax.experimental.pallas import tpu as pltpu
from jax.experimental.pallas import tpu_sc as plsc

G = 8
GS = 12
LANES = 16
ROWS = 8
NBUF = 3


def _make_sc_kernel(n_rows, h, w):
    info = plsc.get_sparse_core_info()
    nc = info.num_cores
    n_chunks = h // ROWS
    assert n_chunks % NBUF == 0
    mesh = plsc.VectorSubcoreMesh(core_axis_name="c", subcore_axis_name="s")

    @functools.partial(
        pl.kernel,
        mesh=mesh,
        out_type=jax.ShapeDtypeStruct((n_rows, GS, h, w), jnp.float32),
        scratch_types=[
            pltpu.VMEM((GS, ROWS, w), jnp.float32),
            pltpu.VMEM((GS, ROWS, w), jnp.float32),
            pltpu.VMEM((GS, ROWS, w), jnp.float32),
            pltpu.VMEM((2 * GS, LANES), jnp.float32),
            pltpu.SemaphoreType.DMA,
            pltpu.SemaphoreType.DMA,
            pltpu.SemaphoreType.DMA,
            pltpu.SemaphoreType.DMA,
            pltpu.SemaphoreType.DMA,
            pltpu.SemaphoreType.DMA,
        ],
    )
    def sc_kernel(x_hbm, w_hbm, out_hbm, bf0, bf1, bf2, w_v,
                  si0, si1, si2, so0, so1, so2):
        wid = lax.axis_index("s") * nc + lax.axis_index("c")
        g = lax.rem(wid, G)
        pltpu.sync_copy(w_hbm.at[g], w_v)
        w0 = [w_v[j] for j in range(GS)]
        w1 = [w_v[GS + j] for j in range(GS)]
        bufs = (bf0, bf1, bf2)
        sin, sout = (si0, si1, si2), (so0, so1, so2)

        def in_slice(ci):
            return x_hbm.at[wid, :, pl.ds(ci * ROWS, ROWS), :]

        def out_slice(ci):
            return out_hbm.at[wid, :, pl.ds(ci * ROWS, ROWS), :]

        pltpu.make_async_copy(in_slice(0), bf0, si0).start()
        pltpu.make_async_copy(in_slice(1), bf1, si1).start()

        def step(i, carry):
            for b in range(NBUF):
                ci = NBUF * i + b
                buf = bufs[b]
                nb = (b + 2) % NBUF

                pltpu.make_async_copy(in_slice(ci), buf, sin[b]).wait()

                @plsc.parallel_loop(0, w // LANES, unroll=2)
                def pix_body(p):
                    po = p * LANES
                    for r in range(ROWS):
                        vals = [buf[j, r, pl.ds(po, LANES)] for j in range(GS)]
                        m1 = jnp.maximum(vals[0], vals[1])
                        m2 = jnp.minimum(vals[0], vals[1])
                        for j in range(2, GS):
                            v = vals[j]
                            m2 = jnp.maximum(m2, jnp.minimum(m1, v))
                            m1 = jnp.maximum(m1, v)
                        for j in range(GS):
                            buf[j, r, pl.ds(po, LANES)] = (
                                vals[j] + w0[j] * m1 + w1[j] * m2)

                pltpu.make_async_copy(buf, out_slice(ci), sout[b]).start()

                @pl.when(ci >= 1)
                def _drain_prev():
                    pltpu.make_async_copy(
                        bufs[nb], out_slice(ci - 1), sout[nb]).wait()

                @pl.when(ci + 2 < n_chunks)
                def _next_in():
                    pltpu.make_async_copy(
                        in_slice(ci + 2), bufs[nb], sin[nb]).start()
            return carry

        lax.fori_loop(0, n_chunks // NBUF, step, 0)
        pltpu.make_async_copy(
            bufs[(n_chunks - 1) % NBUF], out_slice(n_chunks - 1),
            sout[(n_chunks - 1) % NBUF]).wait()

    return sc_kernel


def kernel(input_tensor, weight):
    n, c, h, w = input_tensor.shape
    x4 = input_tensor.reshape(n * G, GS, h, w)
    wr = weight.reshape(G, GS, 2)
    wcat = jnp.concatenate([wr[:, :, 0], wr[:, :, 1]], axis=1)
    wb = jnp.broadcast_to(wcat[:, :, None], (G, 2 * GS, LANES))
    out4 = _make_sc_kernel(n * G, h, w)(x4, wb)
    return out4.reshape(n, c, h, w)


# speedup vs baseline: 4.7869x; 1.0935x over previous
import functools

import jax
import jax.numpy as jnp
from jax import lax
from jax.experimental import pallas as pl
from jax.experimental.pallas import tpu as pltpu
from jax.experimental.pallas import tpu_sc as plsc

G = 8
GS = 12
LANES = 16
ROWS = 8
NBUF = 3


def _make_sc_kernel(n_rows, h, w):
    info = plsc.get_sparse_core_info()
    nc = info.num_cores
    n_chunks = h // ROWS
    assert n_chunks % NBUF == 0
    mesh = plsc.VectorSubcoreMesh(core_axis_name="c", subcore_axis_name="s")

    @functools.partial(
        pl.kernel,
        mesh=mesh,
        out_type=jax.ShapeDtypeStruct((n_rows, GS, h, w), jnp.float32),
        scratch_types=[
            pltpu.VMEM((GS, ROWS, w), jnp.float32),
            pltpu.VMEM((GS, ROWS, w), jnp.float32),
            pltpu.VMEM((GS, ROWS, w), jnp.float32),
            pltpu.VMEM((2 * GS, LANES), jnp.float32),
            pltpu.SemaphoreType.DMA,
            pltpu.SemaphoreType.DMA,
            pltpu.SemaphoreType.DMA,
            pltpu.SemaphoreType.DMA,
            pltpu.SemaphoreType.DMA,
            pltpu.SemaphoreType.DMA,
        ],
    )
    def sc_kernel(x_hbm, w_hbm, out_hbm, bf0, bf1, bf2, w_v,
                  si0, si1, si2, so0, so1, so2):
        wid = lax.axis_index("s") * nc + lax.axis_index("c")
        g = lax.rem(wid, G)
        pltpu.sync_copy(w_hbm.at[g], w_v)
        w0 = [w_v[j] for j in range(GS)]
        w1 = [w_v[GS + j] for j in range(GS)]
        bufs = (bf0, bf1, bf2)
        sin, sout = (si0, si1, si2), (so0, so1, so2)

        def in_slice(ci):
            return x_hbm.at[wid, :, pl.ds(ci * ROWS, ROWS), :]

        def out_slice(ci):
            return out_hbm.at[wid, :, pl.ds(ci * ROWS, ROWS), :]

        pltpu.make_async_copy(in_slice(0), bf0, si0).start()
        pltpu.make_async_copy(in_slice(1), bf1, si1).start()

        def step(i, carry):
            for b in range(NBUF):
                ci = NBUF * i + b
                buf = bufs[b]
                nb = (b + 2) % NBUF

                pltpu.make_async_copy(in_slice(ci), buf, sin[b]).wait()

                @plsc.parallel_loop(0, w // LANES, unroll=2)
                def pix_body(p):
                    po = p * LANES
                    for r in range(ROWS):
                        vals = [buf[j, r, pl.ds(po, LANES)] for j in range(GS)]
                        m1 = jnp.maximum(vals[0], vals[1])
                        m2 = jnp.minimum(vals[0], vals[1])
                        for j in range(2, GS):
                            v = vals[j]
                            m2 = jnp.maximum(m2, jnp.minimum(m1, v))
                            m1 = jnp.maximum(m1, v)
                        for j in range(GS):
                            plsc.addupdate(buf.at[j, r, pl.ds(po, LANES)],
                                           w0[j] * m1 + w1[j] * m2)

                pltpu.make_async_copy(buf, out_slice(ci), sout[b]).start()

                @pl.when(ci >= 1)
                def _drain_prev():
                    pltpu.make_async_copy(
                        bufs[nb], out_slice(ci - 1), sout[nb]).wait()

                @pl.when(ci + 2 < n_chunks)
                def _next_in():
                    pltpu.make_async_copy(
                        in_slice(ci + 2), bufs[nb], sin[nb]).start()
            return carry

        lax.fori_loop(0, n_chunks // NBUF, step, 0)
        pltpu.make_async_copy(
            bufs[(n_chunks - 1) % NBUF], out_slice(n_chunks - 1),
            sout[(n_chunks - 1) % NBUF]).wait()

    return sc_kernel


def kernel(input_tensor, weight):
    n, c, h, w = input_tensor.shape
    x4 = input_tensor.reshape(n * G, GS, h, w)
    wr = weight.reshape(G, GS, 2)
    wcat = jnp.concatenate([wr[:, :, 0], wr[:, :, 1]], axis=1)
    wb = jnp.broadcast_to(wcat[:, :, None], (G, 2 * GS, LANES))
    out4 = _make_sc_kernel(n * G, h, w)(x4, wb)
    return out4.reshape(n, c, h, w)
